# baseline (speedup 1.0000x reference)
import jax
import jax.numpy as jnp
from jax.experimental import pallas as pl
from jax.experimental.pallas import tpu as pltpu

F32 = jnp.float32
BF16 = jnp.bfloat16

D = 128
KEIG = 256
NCLS = 9
N0, N1, N2 = 2048, 6144, 4096

BP1 = 256
BP2 = 256
NP1 = N0 // BP1
NP2 = N1 // BP2

P1_LO = 1
P1_HI = P1_LO + NP1 - 1
P2_LO = P1_HI + 1
P2_HI = P2_LO + NP2 - 1
P3 = P2_HI + 1
P4_LO = P3 + 1
P4_HI = P4_LO + NP1 - 1
P5 = P4_HI + 1
NSTEPS = P5 + 1


def _dot(a, b):
    return jax.lax.dot_general(a, b, (((1,), (0,)), ((), ())),
                               preferred_element_type=F32)


def _dot_tn(a, b):
    return jax.lax.dot_general(a, b, (((0,), (0,)), ((), ())),
                               preferred_element_type=F32)


def _net_body(x0, x1, x2, e0, ed1, eu1, evs,
              wpk, wout, bout,
              inc1, inc2,
              out,
              xw01, xw10, xw21, y0m, y1acc, x0n, s0s, s1s, y0mb):
    s = pl.program_id(0)

    @pl.when(s == 0)
    def _prep():
        x0v = x0[...]
        x1v = x1[...]
        xw01[...] = _dot(x0v, wpk[13]).astype(BF16)
        xw10[...] = _dot(x1v, wpk[6]).astype(BF16)
        xw21[...] = _dot(x2[...], wpk[14]).astype(BF16)
        x1b = x1v.astype(BF16)
        t0 = _dot_tn(e0[...].astype(BF16), x0v.astype(BF16))
        td = _dot_tn(ed1[...].astype(BF16), x1b)
        tu = _dot_tn(eu1[...].astype(BF16), x1b)
        e0v = evs[0]
        s0s[...] = (_dot(e0v * t0, wpk[1])
                    + _dot(e0v * e0v * t0, wpk[2])).astype(BF16)
        ed = evs[1]
        eu = evs[2]
        s1s[0:KEIG, :] = (_dot(ed * td, wpk[9])
                          + _dot(ed * ed * td, wpk[10])).astype(BF16)
        s1s[KEIG:2 * KEIG, :] = (_dot(eu * tu, wpk[11])
                                 + _dot(eu * eu * tu, wpk[12])).astype(BF16)

    @pl.when((s >= P1_LO) & (s <= P1_HI))
    def _pass1():
        i = s - P1_LO
        inc = inc1[...].astype(BF16)
        y0m[pl.ds(i * BP1, BP1), :] = _dot(inc, xw10[...])
        b = _dot_tn(inc, xw01[pl.ds(i * BP1, BP1), :])

        @pl.when(i == 0)
        def _():
            y1acc[...] = b

        @pl.when(i > 0)
        def _():
            y1acc[...] += b

    @pl.when((s >= P2_LO) & (s <= P2_HI))
    def _pass2():
        i = s - P2_LO
        inc = inc2[...].astype(BF16)
        y1acc[pl.ds(i * BP2, BP2), :] += _dot(inc, xw21[...])

    @pl.when(s == P3)
    def _combine0():
        x0v = x0[...]
        x1v = x1[...]
        y0 = (_dot(x0v, wpk[0]) + y0m[...]
              + _dot(e0[...].astype(BF16), s0s[...]))
        x0nv = jax.nn.sigmoid(y0)
        x0n[...] = x0nv
        y1 = (_dot(x1v, wpk[8]) + y1acc[...]
              + _dot(ed1[...].astype(BF16), s1s[0:KEIG, :])
              + _dot(eu1[...].astype(BF16), s1s[KEIG:2 * KEIG, :]))
        x1nv = jax.nn.sigmoid(y1)
        xw10[...] = _dot(x1nv, wpk[7]).astype(BF16)
        t0b = _dot_tn(e0[...].astype(BF16), x0nv.astype(BF16))
        e0v = evs[0]
        s0s[...] = (_dot(e0v * t0b, wpk[4])
                    + _dot(e0v * e0v * t0b, wpk[5])).astype(BF16)

    @pl.when((s >= P4_LO) & (s <= P4_HI))
    def _pass4():
        i = s - P4_LO
        inc = inc1[...].astype(BF16)
        y0mb[pl.ds(i * BP1, BP1), :] = _dot(inc, xw10[...])

    @pl.when(s == P5)
    def _head():
        y0 = (_dot(x0n[...], wpk[3]) + y0mb[...]
              + _dot(e0[...].astype(BF16), s0s[...]))
        x0f = jax.nn.sigmoid(y0)
        out[...] = _dot(x0f, wout[...]) + bout[...]


def _inc1_map(s):
    sa = jnp.clip(s, P1_LO, P1_HI) - P1_LO
    sb = jnp.clip(s, P4_LO, P4_HI) - P4_LO
    return (jnp.where(s >= P4_LO, sb, sa), 0)


def _inc2_map(s):
    return (jnp.clip(s, P2_LO, P2_HI) - P2_LO, 0)


def _whole(*shape):
    return pl.BlockSpec(shape, lambda s: (0,) * len(shape))


def kernel(x_0, x_1, x_2, evals_0, evecs_0, evals_d1, evecs_d1, evals_u1,
           evecs_u1, evals_d2, evecs_d2, evals_u2, evecs_u2, incidence_1,
           incidence_2, W0, W10, W1id, W1d, W1u, W01, W21, W2id, W2d, W2u,
           W12, Wout, bout):
    evs = jnp.stack([evals_0, evals_d1, evals_u1], axis=0).reshape(3, KEIG, 1)
    wpk = jnp.concatenate([
        W0.reshape(6, D, D),
        W10, W1id[0:1], W1d[0], W1u[0], W01[0:1], W21[0:1]], axis=0)

    in_specs = [
        _whole(N0, D), _whole(N1, D), _whole(N2, D),
        _whole(N0, KEIG), _whole(N1, KEIG), _whole(N1, KEIG),
        _whole(3, KEIG, 1),
        _whole(15, D, D),
        _whole(D, NCLS), pl.BlockSpec((NCLS,), lambda s: (0,)),
        pl.BlockSpec((BP1, N1), _inc1_map),
        pl.BlockSpec((BP2, N2), _inc2_map),
    ]
    scratch = [
        pltpu.VMEM((N0, D), BF16),
        pltpu.VMEM((N1, D), BF16),
        pltpu.VMEM((N2, D), BF16),
        pltpu.VMEM((N0, D), F32),
        pltpu.VMEM((N1, D), F32),
        pltpu.VMEM((N0, D), F32),
        pltpu.VMEM((KEIG, D), BF16),
        pltpu.VMEM((2 * KEIG, D), BF16),
        pltpu.VMEM((N0, D), F32),
    ]
    return pl.pallas_call(
        _net_body,
        grid=(NSTEPS,),
        in_specs=in_specs,
        out_specs=_whole(N0, NCLS),
        out_shape=jax.ShapeDtypeStruct((N0, NCLS), F32),
        scratch_shapes=scratch,
        compiler_params=pltpu.CompilerParams(
            vmem_limit_bytes=63 * 1024 * 1024),
    )(x_0, x_1, x_2, evecs_0, evecs_d1, evecs_u1, evs,
      wpk, Wout, bout,
      incidence_1, incidence_2)
